# Initial kernel scaffold; baseline (speedup 1.0000x reference)
import jax
import jax.numpy as jnp
from jax.experimental import pallas as pl


def kernel(nodes, history_uv, history_r, v2e_w, u2e_w, r2e_w, w_r1_w, w_r1_b, w_r2_w, w_r2_b, att1_w, att1_b, att2_w, att2_b, att3_w, att3_b):
    raise NotImplementedError("write your pallas kernel here")


# speedup vs baseline: 1.9127x; 1.9127x over previous
import functools

import jax
import jax.numpy as jnp
from jax import lax
from jax.experimental import pallas as pl
from jax.experimental.pallas import tpu as pltpu
from jax.experimental.pallas import tpu_sc as plsc

B, L, V, R, D = 1024, 50, 100000, 5, 64
LP = 56
NT = B * LP
NW = 32
CHUNK = 128
TPW = NT // NW
CPW = TPW // CHUNK
NPW = B // NW


def _sc_gather(hist_idx, nodes, v2e_w, u2e_w):
    mesh = plsc.VectorSubcoreMesh(core_axis_name="c", subcore_axis_name="s")

    @functools.partial(
        pl.kernel,
        mesh=mesh,
        compiler_params=pltpu.CompilerParams(use_tc_tiling_on_sc=False),
        out_type=[
            jax.ShapeDtypeStruct((NT // CHUNK, CHUNK, D), jnp.float32),
            jax.ShapeDtypeStruct((B, D), jnp.float32),
        ],
        scratch_types=[
            pltpu.VMEM((CPW, CHUNK), jnp.int32),
            pltpu.VMEM((CPW, CHUNK, D), jnp.float32),
            pltpu.VMEM((NPW,), jnp.int32),
            pltpu.VMEM((NPW, D), jnp.float32),
            pltpu.SemaphoreType.DMA,
            pltpu.SemaphoreType.DMA,
        ],
    )
    def gather_kernel(v2e_hbm, u2e_hbm, hist_hbm, nodes_hbm,
                      euv_out, urep_out, idx_v, rows_v, nidx_v, nrows_v,
                      sem, nsem):
        wid = lax.axis_index("s") * 2 + lax.axis_index("c")
        base = wid * CPW
        pltpu.sync_copy(hist_hbm.at[wid], idx_v)
        nbase = wid * NPW
        pltpu.sync_copy(nodes_hbm.at[pl.ds(nbase, NPW)], nidx_v)
        copies = []
        for c in range(CPW):
            copies.append(
                pltpu.async_copy(v2e_hbm.at[idx_v.at[c]], rows_v.at[c], sem))
        ncopy = pltpu.async_copy(u2e_hbm.at[nidx_v], nrows_v, nsem)
        for cp in copies:
            cp.wait()
        pltpu.sync_copy(rows_v, euv_out.at[pl.ds(base, CPW)])
        ncopy.wait()
        pltpu.sync_copy(nrows_v, urep_out.at[pl.ds(nbase, NPW)])

    return gather_kernel(v2e_w, u2e_w, hist_idx, nodes)


BB = 128
NTOK = BB * LP


def _dense_body(euv_ref, urep_ref, hr_ref,
                w1a_ref, cr_ref, w2_ref, b2_ref,
                a1a_ref, a1b_ref, a1bias_ref, a2_ref, a2b_ref, att3_ref,
                out_ref):
    euv = euv_ref[...]
    hr = hr_ref[...]
    onehot3 = (hr[:, :, None] == lax.broadcasted_iota(jnp.int32, (1, 1, 8), 2))
    onehot = onehot3.astype(jnp.float32).reshape(NTOK, 8)
    contrib = jnp.dot(onehot, cr_ref[...],
                      preferred_element_type=jnp.float32)
    x1 = jnp.maximum(jnp.dot(euv, w1a_ref[...],
                             preferred_element_type=jnp.float32) + contrib, 0.0)
    o = jnp.maximum(jnp.dot(x1, w2_ref[...],
                            preferred_element_type=jnp.float32) + b2_ref[...], 0.0)
    u_att = jnp.dot(urep_ref[...], a1b_ref[...],
                    preferred_element_type=jnp.float32) + a1bias_ref[...]
    u_att_tok = jnp.broadcast_to(u_att[:, None, :], (BB, LP, D)).reshape(NTOK, D)
    a1 = jnp.maximum(jnp.dot(o, a1a_ref[...],
                             preferred_element_type=jnp.float32) + u_att_tok, 0.0)
    a2 = jnp.maximum(jnp.dot(a1, a2_ref[...],
                             preferred_element_type=jnp.float32) + a2b_ref[...], 0.0)
    a2_3d = a2.reshape(BB, LP, D)
    logits = jnp.sum(a2_3d * att3_ref[...][None, :, :], axis=2)
    lmask = lax.broadcasted_iota(jnp.int32, (BB, LP), 1) < L
    logits = jnp.where(lmask, logits, -jnp.inf)
    m = jnp.max(logits, axis=1, keepdims=True)
    e = jnp.exp(logits - m)
    w = e / jnp.sum(e, axis=1, keepdims=True)
    o_3d = o.reshape(BB, LP, D)
    out_ref[...] = jnp.sum(o_3d * w[:, :, None], axis=1)


def _dense(e_uv, u_rep, hr_pad, w1a_t, c_r, w2_t, b2,
           a1a_t, a1b_t, a1bias, a2_t, a2b, att3v):
    grid = B // BB
    full = lambda shape: pl.BlockSpec(shape, lambda i: (0,) * len(shape))
    return pl.pallas_call(
        _dense_body,
        grid=(grid,),
        in_specs=[
            pl.BlockSpec((NTOK, D), lambda i: (i, 0)),
            pl.BlockSpec((BB, D), lambda i: (i, 0)),
            pl.BlockSpec((BB, LP), lambda i: (i, 0)),
            full((D, D)),
            full((8, D)),
            full((D, D)),
            full((1, D)),
            full((D, D)),
            full((D, D)),
            full((1, D)),
            full((D, D)),
            full((1, D)),
            full((1, D)),
        ],
        out_specs=pl.BlockSpec((BB, D), lambda i: (i, 0)),
        out_shape=jax.ShapeDtypeStruct((B, D), jnp.float32),
        compiler_params=pltpu.CompilerParams(
            dimension_semantics=("arbitrary",)),
    )(e_uv, u_rep, hr_pad, w1a_t, c_r, w2_t, b2,
      a1a_t, a1b_t, a1bias, a2_t, a2b, att3v)


def kernel(nodes, history_uv, history_r, v2e_w, u2e_w, r2e_w,
           w_r1_w, w_r1_b, w_r2_w, w_r2_b,
           att1_w, att1_b, att2_w, att2_b, att3_w, att3_b):
    w1a_t = w_r1_w[:, :D].T
    c_r = r2e_w @ w_r1_w[:, D:].T + w_r1_b
    c_r = jnp.pad(c_r, ((0, 8 - R), (0, 0)))
    w2_t = w_r2_w.T
    b2 = w_r2_b[None, :]
    a1a_t = att1_w[:, :D].T
    a1b_t = att1_w[:, D:].T
    a1bias = att1_b[None, :]
    a2_t = att2_w.T
    a2b = att2_b[None, :]
    att3v = att3_w

    hist_pad = jnp.pad(history_uv, ((0, 0), (0, LP - L)))
    hist_idx = hist_pad.reshape(NW, CPW, CHUNK)
    hr_pad = jnp.pad(history_r, ((0, 0), (0, LP - L)))

    e_uv3, u_rep = _sc_gather(hist_idx, nodes, v2e_w, u2e_w)
    e_uv = e_uv3.reshape(NT, D)

    return _dense(e_uv, u_rep, hr_pad, w1a_t, c_r, w2_t, b2,
                  a1a_t, a1b_t, a1bias, a2_t, a2b, att3v)


# speedup vs baseline: 1.9166x; 1.0020x over previous
import functools

import jax
import jax.numpy as jnp
from jax import lax
from jax.experimental import pallas as pl
from jax.experimental.pallas import tpu as pltpu
from jax.experimental.pallas import tpu_sc as plsc

B, L, V, R, D = 1024, 50, 100000, 5, 64
LP = 56
NT = B * LP
NW = 32
CHUNK = 128
TPW = NT // NW
CPW = TPW // CHUNK
NPW = B // NW


def _sc_gather(hist_idx, nodes, v2e_w, u2e_w):
    mesh = plsc.VectorSubcoreMesh(core_axis_name="c", subcore_axis_name="s")

    @functools.partial(
        pl.kernel,
        mesh=mesh,
        compiler_params=pltpu.CompilerParams(use_tc_tiling_on_sc=False),
        out_type=[
            jax.ShapeDtypeStruct((NT, D), jnp.float32),
            jax.ShapeDtypeStruct((B, D), jnp.float32),
        ],
        scratch_types=[
            pltpu.VMEM((TPW,), jnp.int32),
            pltpu.VMEM((TPW, D), jnp.float32),
            pltpu.VMEM((NPW,), jnp.int32),
            pltpu.VMEM((NPW, D), jnp.float32),
            pltpu.SemaphoreType.DMA,
            pltpu.SemaphoreType.DMA,
        ],
    )
    def gather_kernel(v2e_hbm, u2e_hbm, hist_hbm, nodes_hbm,
                      euv_out, urep_out, idx_v, rows_v, nidx_v, nrows_v,
                      sem, nsem):
        wid = lax.axis_index("s") * 2 + lax.axis_index("c")
        base = wid * TPW
        pltpu.sync_copy(hist_hbm.at[pl.ds(base, TPW)], idx_v)
        nbase = wid * NPW
        pltpu.sync_copy(nodes_hbm.at[pl.ds(nbase, NPW)], nidx_v)
        cp = pltpu.async_copy(v2e_hbm.at[idx_v], rows_v, sem)
        ncopy = pltpu.async_copy(u2e_hbm.at[nidx_v], nrows_v, nsem)
        cp.wait()
        pltpu.sync_copy(rows_v, euv_out.at[pl.ds(base, TPW)])
        ncopy.wait()
        pltpu.sync_copy(nrows_v, urep_out.at[pl.ds(nbase, NPW)])

    return gather_kernel(v2e_w, u2e_w, hist_idx, nodes)


BB = 128
NTOK = BB * LP


def _dense_body(euv_ref, urep_ref, hr_ref,
                w1a_ref, cr_ref, w2_ref, b2_ref,
                a1a_ref, a1b_ref, a1bias_ref, a2_ref, a2b_ref, att3_ref,
                out_ref):
    euv = euv_ref[...]
    hr = hr_ref[...]
    onehot3 = (hr[:, :, None] == lax.broadcasted_iota(jnp.int32, (1, 1, 8), 2))
    onehot = onehot3.astype(jnp.float32).reshape(NTOK, 8)
    contrib = jnp.dot(onehot, cr_ref[...],
                      preferred_element_type=jnp.float32)
    x1 = jnp.maximum(jnp.dot(euv, w1a_ref[...],
                             preferred_element_type=jnp.float32) + contrib, 0.0)
    o = jnp.maximum(jnp.dot(x1, w2_ref[...],
                            preferred_element_type=jnp.float32) + b2_ref[...], 0.0)
    u_att = jnp.dot(urep_ref[...], a1b_ref[...],
                    preferred_element_type=jnp.float32) + a1bias_ref[...]
    u_att_tok = jnp.broadcast_to(u_att[:, None, :], (BB, LP, D)).reshape(NTOK, D)
    a1 = jnp.maximum(jnp.dot(o, a1a_ref[...],
                             preferred_element_type=jnp.float32) + u_att_tok, 0.0)
    a2 = jnp.maximum(jnp.dot(a1, a2_ref[...],
                             preferred_element_type=jnp.float32) + a2b_ref[...], 0.0)
    a2_3d = a2.reshape(BB, LP, D)
    logits = jnp.sum(a2_3d * att3_ref[...][None, :, :], axis=2)
    lmask = lax.broadcasted_iota(jnp.int32, (BB, LP), 1) < L
    logits = jnp.where(lmask, logits, -jnp.inf)
    m = jnp.max(logits, axis=1, keepdims=True)
    e = jnp.exp(logits - m)
    w = e / jnp.sum(e, axis=1, keepdims=True)
    o_3d = o.reshape(BB, LP, D)
    out_ref[...] = jnp.sum(o_3d * w[:, :, None], axis=1)


def _dense(e_uv, u_rep, hr_pad, w1a_t, c_r, w2_t, b2,
           a1a_t, a1b_t, a1bias, a2_t, a2b, att3v):
    grid = B // BB
    full = lambda shape: pl.BlockSpec(shape, lambda i: (0,) * len(shape))
    return pl.pallas_call(
        _dense_body,
        grid=(grid,),
        in_specs=[
            pl.BlockSpec((NTOK, D), lambda i: (i, 0)),
            pl.BlockSpec((BB, D), lambda i: (i, 0)),
            pl.BlockSpec((BB, LP), lambda i: (i, 0)),
            full((D, D)),
            full((8, D)),
            full((D, D)),
            full((1, D)),
            full((D, D)),
            full((D, D)),
            full((1, D)),
            full((D, D)),
            full((1, D)),
            full((1, D)),
        ],
        out_specs=pl.BlockSpec((BB, D), lambda i: (i, 0)),
        out_shape=jax.ShapeDtypeStruct((B, D), jnp.float32),
        compiler_params=pltpu.CompilerParams(
            dimension_semantics=("arbitrary",)),
    )(e_uv, u_rep, hr_pad, w1a_t, c_r, w2_t, b2,
      a1a_t, a1b_t, a1bias, a2_t, a2b, att3v)


def kernel(nodes, history_uv, history_r, v2e_w, u2e_w, r2e_w,
           w_r1_w, w_r1_b, w_r2_w, w_r2_b,
           att1_w, att1_b, att2_w, att2_b, att3_w, att3_b):
    w1a_t = w_r1_w[:, :D].T
    c_r = r2e_w @ w_r1_w[:, D:].T + w_r1_b
    c_r = jnp.pad(c_r, ((0, 8 - R), (0, 0)))
    w2_t = w_r2_w.T
    b2 = w_r2_b[None, :]
    a1a_t = att1_w[:, :D].T
    a1b_t = att1_w[:, D:].T
    a1bias = att1_b[None, :]
    a2_t = att2_w.T
    a2b = att2_b[None, :]
    att3v = att3_w

    hist_pad = jnp.pad(history_uv, ((0, 0), (0, LP - L)))
    hist_idx = hist_pad.reshape(NT)
    hr_pad = jnp.pad(history_r, ((0, 0), (0, LP - L)))

    e_uv, u_rep = _sc_gather(hist_idx, nodes, v2e_w, u2e_w)

    return _dense(e_uv, u_rep, hr_pad, w1a_t, c_r, w2_t, b2,
                  a1a_t, a1b_t, a1bias, a2_t, a2b, att3v)
